# baseline (speedup 1.0000x reference)
import functools

import jax
import jax.numpy as jnp
from jax import lax
from jax.experimental import pallas as pl
from jax.experimental.pallas import tpu as pltpu, tpu_sc as plsc

_B, _T, _C = 4, 4096, 2048
_E = 8
_K = 2
_N = _B * _T

_ROWS = 256
_NBUF = 8
_NCHUNK = _N // _ROWS


def _topk2(p):
    rows = p.shape[0]
    iota_e = lax.broadcasted_iota(jnp.int32, (rows, _E), 1)
    m1 = jnp.max(p, axis=-1, keepdims=True)
    i1 = jnp.min(jnp.where(p == m1, iota_e, _E), axis=-1, keepdims=True)
    p2 = jnp.where(iota_e == i1, -jnp.inf, p)
    m2 = jnp.max(p2, axis=-1, keepdims=True)
    i2 = jnp.min(jnp.where(p2 == m2, iota_e, _E), axis=-1, keepdims=True)
    return m1, i1, m2, i2


def _router_body(x_hbm, w_ref, b_ref, op_ref, otp_ref, oti_ref, buf, sem):
    i = pl.program_id(0)
    slot = lax.rem(i, _NBUF)

    @pl.when(i == 0)
    def _prime():
        for k in range(_NBUF):
            pltpu.make_async_copy(
                x_hbm.at[pl.ds(k * _ROWS, _ROWS), :], buf.at[k], sem.at[k]
            ).start()

    nxt = i + _NBUF - 1
    prev_slot = lax.rem(nxt, _NBUF)

    @pl.when(jnp.logical_and(i > 0, nxt < _NCHUNK))
    def _refill():
        pltpu.make_async_copy(
            x_hbm.at[pl.ds(nxt * _ROWS, _ROWS), :],
            buf.at[prev_slot],
            sem.at[prev_slot],
        ).start()

    pltpu.make_async_copy(
        x_hbm.at[pl.ds(i * _ROWS, _ROWS), :], buf.at[slot], sem.at[slot]
    ).wait()

    g = jnp.dot(buf[slot], w_ref[...], preferred_element_type=jnp.float32)
    g = g + b_ref[...]
    m = jnp.max(g, axis=-1, keepdims=True)
    e = jnp.exp(g - m)
    p = e / jnp.sum(e, axis=-1, keepdims=True)
    op_ref[...] = p
    m1, i1, m2, i2 = _topk2(p)
    otp_ref[...] = jnp.concatenate([m1, m2], axis=-1)
    oti_ref[...] = jnp.concatenate([i1, i2], axis=-1)


def _router(x2, w, b2):
    return pl.pallas_call(
        _router_body,
        grid=(_NCHUNK,),
        in_specs=[
            pl.BlockSpec(memory_space=pl.ANY),
            pl.BlockSpec((_C, _E), lambda i: (0, 0)),
            pl.BlockSpec((1, _E), lambda i: (0, 0)),
        ],
        out_specs=[
            pl.BlockSpec((_ROWS, _E), lambda i: (i, 0)),
            pl.BlockSpec((_ROWS, _K), lambda i: (i, 0)),
            pl.BlockSpec((_ROWS, _K), lambda i: (i, 0)),
        ],
        out_shape=[
            jax.ShapeDtypeStruct((_N, _E), jnp.float32),
            jax.ShapeDtypeStruct((_N, _K), jnp.float32),
            jax.ShapeDtypeStruct((_N, _K), jnp.int32),
        ],
        scratch_shapes=[
            pltpu.VMEM((_NBUF, _ROWS, _C), jnp.float32),
            pltpu.SemaphoreType.DMA((_NBUF,)),
        ],
    )(x2, w, b2)


def kernel(x, router_w, router_b):
    x2 = x.reshape(_N, _C)
    gate_probs, top_p, top_i = _router(x2, router_w, router_b.reshape(1, _E))
    return (
        gate_probs.reshape(_B, _T, _E),
        top_p.reshape(_B, _T, _K),
        top_i.reshape(_B, _T, _K),
    )


# speedup vs baseline: 1.3761x; 1.3761x over previous
import functools

import jax
import jax.numpy as jnp
from jax import lax
from jax.experimental import pallas as pl
from jax.experimental.pallas import tpu as pltpu, tpu_sc as plsc

_B, _T, _C = 4, 4096, 2048
_E = 8
_K = 2
_N = _B * _T

_ROWS = 512
_NBUF = 8
_NCHUNK = _N // _ROWS


def _router_body(x_hbm, w_ref, b_ref, op_ref, otp_ref, oti_ref, buf, sem):
    i = pl.program_id(0)
    slot = lax.rem(i, _NBUF)

    @pl.when(i == 0)
    def _prime():
        for k in range(_NBUF):
            pltpu.make_async_copy(
                x_hbm.at[pl.ds(k * _ROWS, _ROWS), :], buf.at[k], sem.at[k]
            ).start()

    nxt = i + _NBUF - 1
    prev_slot = lax.rem(nxt, _NBUF)

    @pl.when(jnp.logical_and(i > 0, nxt < _NCHUNK))
    def _refill():
        pltpu.make_async_copy(
            x_hbm.at[pl.ds(nxt * _ROWS, _ROWS), :],
            buf.at[prev_slot],
            sem.at[prev_slot],
        ).start()

    pltpu.make_async_copy(
        x_hbm.at[pl.ds(i * _ROWS, _ROWS), :], buf.at[slot], sem.at[slot]
    ).wait()

    g = jnp.dot(buf[slot], w_ref[...], preferred_element_type=jnp.float32)
    g = g + b_ref[...]
    m = jnp.max(g, axis=-1, keepdims=True)
    e = jnp.exp(g - m)
    r = 1.0 / jnp.sum(e, axis=-1, keepdims=True)
    p = e * r
    op_ref[...] = p
    iota_e = lax.broadcasted_iota(jnp.int32, (_ROWS, _E), 1).astype(jnp.float32)
    i1 = jnp.min(jnp.where(e == 1.0, iota_e, 8.0), axis=-1, keepdims=True)
    e2 = jnp.where(iota_e == i1, -1.0, e)
    m2 = jnp.max(e2, axis=-1, keepdims=True)
    i2 = jnp.min(jnp.where(e2 == m2, iota_e, 8.0), axis=-1, keepdims=True)
    otp_ref[...] = jnp.concatenate([r, m2 * r], axis=-1)
    oti_ref[...] = jnp.concatenate([i1, i2], axis=-1).astype(jnp.int32)


def _router(x2, w, b2):
    return pl.pallas_call(
        _router_body,
        grid=(_NCHUNK,),
        in_specs=[
            pl.BlockSpec(memory_space=pl.ANY),
            pl.BlockSpec((_C, _E), lambda i: (0, 0)),
            pl.BlockSpec((1, _E), lambda i: (0, 0)),
        ],
        out_specs=[
            pl.BlockSpec((_ROWS, _E), lambda i: (i, 0)),
            pl.BlockSpec((_ROWS, _K), lambda i: (i, 0)),
            pl.BlockSpec((_ROWS, _K), lambda i: (i, 0)),
        ],
        out_shape=[
            jax.ShapeDtypeStruct((_N, _E), jnp.float32),
            jax.ShapeDtypeStruct((_N, _K), jnp.float32),
            jax.ShapeDtypeStruct((_N, _K), jnp.int32),
        ],
        scratch_shapes=[
            pltpu.VMEM((_NBUF, _ROWS, _C), jnp.float32),
            pltpu.SemaphoreType.DMA((_NBUF,)),
        ],
    )(x2, w, b2)


def kernel(x, router_w, router_b):
    x2 = x.reshape(_N, _C)
    gate_probs, top_p, top_i = _router(x2, router_w, router_b.reshape(1, _E))
    return (
        gate_probs.reshape(_B, _T, _E),
        top_p.reshape(_B, _T, _K),
        top_i.reshape(_B, _T, _K),
    )
